# baseline (speedup 1.0000x reference)
import functools

import jax
import jax.numpy as jnp
from jax import lax
from jax.experimental import pallas as pl
from jax.experimental.pallas import tpu as pltpu
from jax.experimental.pallas import tpu_sc as plsc

B = 128
N = 32768
L = 16
NCHUNK = N // L
NBINS = 512
NWORKERS = 32
ROWS_PER = B // NWORKERS
U = 8

F32 = jnp.float32
VN = float(jnp.finfo(jnp.float32).min) / 4.0
BIG = float(jnp.finfo(jnp.float32).max)
NEG_INF = float("-inf")
NAN = float("nan")
NACT_NAN_MAX = float(N // 2 - 5)


def _splat(x):
    return jnp.broadcast_to(x, (L,))


def _vsum(v):
    return _splat(jnp.sum(v))


def _mesh():
    return plsc.VectorSubcoreMesh(core_axis_name="c", subcore_axis_name="s")


@functools.partial(
    pl.kernel,
    out_type=jax.ShapeDtypeStruct((B, N), jnp.float32),
    mesh=_mesh(),
    compiler_params=pltpu.CompilerParams(needs_layout_passes=False),
    scratch_types=[
        pltpu.VMEM((N,), jnp.float32),
        pltpu.VMEM((N,), jnp.float32),
        pltpu.VMEM((NBINS * L,), jnp.float32),
        pltpu.VMEM((NBINS * L,), jnp.float32),
        pltpu.VMEM((NBINS,), jnp.float32),
        pltpu.VMEM((NBINS,), jnp.float32),
    ],
)
def _sc_portfolio(logits_hbm, maskf_hbm, out_hbm, zbuf, mcand, hcnt, hsum,
                  tcnt, tsum):
    wid = lax.axis_index("s") * 2 + lax.axis_index("c")
    lane = lax.iota(jnp.int32, L)
    lanef = lane.astype(F32)
    zeros = jnp.full((L,), 0.0, F32)
    ones = jnp.full((L,), 1.0, F32)

    def row_body(r, carry0):
        row = wid * ROWS_PER + r
        pltpu.sync_copy(logits_hbm.at[row], zbuf)
        pltpu.sync_copy(maskf_hbm.at[row], mcand)

        def p1(jj, carry):
            vmax, vmin, vcnt, off = carry
            for u in range(U):
                sl = pl.ds((jj * U + u) * L, L)
                v = zbuf[sl]
                m = mcand[sl]
                act = m > 0.0
                z = jnp.where(act, v, jnp.full((L,), VN, F32))
                zbuf[sl] = z
                c = z > (vmax - 1.0)
                vmax = jnp.maximum(vmax, z)
                vmin = jnp.minimum(vmin,
                                   jnp.where(act, v, jnp.full((L,), BIG, F32)))
                vcnt = vcnt + jnp.where(act, ones, zeros)
                ci = jnp.where(c, jnp.full((L,), 1, jnp.int32),
                               jnp.full((L,), 0, jnp.int32))
                pos = plsc.cumsum(ci)
                idx = pos + (off - 1)
                plsc.store_scatter(mcand, [idx], z, mask=c)
                off = off + jnp.sum(ci)
            return vmax, vmin, vcnt, off

        vmax, vmin, vcnt, nc = lax.fori_loop(
            0, NCHUNK // U, p1,
            (jnp.full((L,), -BIG, F32), jnp.full((L,), BIG, F32), zeros,
             jnp.int32(0)))
        mx = _splat(jnp.max(vmax))
        mn = _splat(jnp.min(vmin))
        nact = _vsum(vcnt)
        nch = (nc + (L - 1)) >> 4

        def nan_row(_):
            nanv = jnp.full((L,), NAN, F32)

            def pn(jj, _c):
                for u in range(U):
                    zbuf[pl.ds((jj * U + u) * L, L)] = nanv
                return 0
            lax.fori_loop(0, NCHUNK // U, pn, 0)
            return 0

        def full_row(_):
            def cand_stats(tau):
                def cs(ch, carry):
                    csum, ccnt = carry
                    v = mcand[pl.ds(ch * L, L)]
                    valid = (lane + ch * L) < nc
                    a = jnp.logical_and(valid, v > tau)
                    csum = csum + jnp.where(a, v, zeros)
                    ccnt = ccnt + jnp.where(a, ones, zeros)
                    return csum, ccnt
                s, c = lax.fori_loop(0, nch, cs, (zeros, zeros))
                return _vsum(s), _vsum(c)

            def newton_cond(st):
                _tau, done, it = st
                return jnp.logical_and(jnp.logical_not(done), it < 64)

            def newton_body(st):
                tau, done, it = st
                s, c = cand_stats(tau)
                t2 = (s - 1.0) / jnp.maximum(c, 1.0)
                return t2, jnp.all(t2 == tau), it + 1

            tau_star, _d, _i = lax.while_loop(
                newton_cond, newton_body,
                (mx - 1.0, jnp.bool_(False), jnp.int32(0)))
            _s_unused, kstd = cand_stats(tau_star)

            k = kstd + (float(N - 4) - nact)
            ireg = k - nact

            def finite_tau(_a):
                def hz(jj, _c):
                    for u in range(U):
                        sl = pl.ds((jj * U + u) * L, L)
                        hcnt[sl] = zeros
                        hsum[sl] = zeros
                    return 0
                lax.fori_loop(0, (NBINS * L) // (L * U), hz, 0)

                w = jnp.where(mx > mn, (mx - mn) * (1.0 / float(NBINS)), ones)
                inv_w = 1.0 / w

                def hb(jj, _c):
                    for u in range(U):
                        v = zbuf[pl.ds((jj * U + u) * L, L)]
                        bf = jnp.clip((v - mn) * inv_w, 0.0, float(NBINS - 1))
                        bi = bf.astype(jnp.int32)
                        idx = bi + lane * NBINS
                        plsc.addupdate_scatter(hcnt, [idx], ones)
                        plsc.addupdate_scatter(hsum, [idx], v)
                    return 0
                lax.fori_loop(0, NCHUNK // U, hb, 0)

                def fold(c, _c):
                    sl = pl.ds(c * L, L)
                    acc_c = zeros
                    acc_s = zeros
                    for l in range(L):
                        acc_c = acc_c + hcnt[pl.ds(l * NBINS + c * L, L)]
                        acc_s = acc_s + hsum[pl.ds(l * NBINS + c * L, L)]
                    tcnt[sl] = acc_c
                    tsum[sl] = acc_s
                    return 0
                lax.fori_loop(0, NBINS // L, fold, 0)

                def sw1(tt, carry):
                    base, jbc = carry
                    c = (NBINS // L - 1) - tt
                    t = tcnt[pl.ds(c * L, L)]
                    tot = _vsum(t)
                    suf = base + tot - plsc.cumsum(t) + t
                    jbc = jbc + jnp.where(suf > k, ones, zeros)
                    return base + tot, jbc
                _b, jbc = lax.fori_loop(0, NBINS // L, sw1, (zeros, zeros))
                jbf = _vsum(jbc)

                def sw2(c, carry):
                    cab, sab = carry
                    binf = lanef + _splat(c * L).astype(F32)
                    above = binf > jbf
                    cab = cab + jnp.where(above, tcnt[pl.ds(c * L, L)], zeros)
                    sab = sab + jnp.where(above, tsum[pl.ds(c * L, L)], zeros)
                    return cab, sab
                cab_v, sab_v = lax.fori_loop(0, NBINS // L, sw2,
                                             (zeros, zeros))
                cab = _vsum(cab_v)
                sab = _vsum(sab_v)

                t_edge = mn + jbf * w
                m_rem = k - cab
                sk = sab + m_rem * t_edge
                return (sk - 1.0) / k

            tau_fin = lax.cond(jnp.all(ireg < 0.5), finite_tau,
                               lambda _a: zeros, 0)
            tau_unif = (ireg * F32(VN) - 1.0) / k
            tau = jnp.where(ireg >= 4.5, jnp.full((L,), NEG_INF, F32),
                            jnp.where(ireg >= 0.5, tau_unif, tau_fin))

            def p4(jj, acc):
                for u in range(U):
                    v = zbuf[pl.ds((jj * U + u) * L, L)]
                    acc = acc + jnp.maximum(v - tau, 0.0)
                return acc
            s1 = _vsum(lax.fori_loop(0, NCHUNK // U, p4, zeros))
            r1 = 1.0 / jnp.maximum(s1, 1e-12)

            def p5(jj, acc):
                for u in range(U):
                    v = zbuf[pl.ds((jj * U + u) * L, L)]
                    wv = jnp.maximum(v - tau, 0.0) * r1
                    acc = acc + jnp.where(wv < 1e-6, zeros, wv)
                return acc
            ws = _vsum(lax.fori_loop(0, NCHUNK // U, p5, zeros))
            r2 = 1.0 / jnp.maximum(ws, 1e-12)

            def p6(jj, _c):
                for u in range(U):
                    sl = pl.ds((jj * U + u) * L, L)
                    v = zbuf[sl]
                    wv = jnp.maximum(v - tau, 0.0) * r1
                    wv = jnp.where(wv < 1e-6, zeros, wv)
                    zbuf[sl] = wv * r2
                return 0
            lax.fori_loop(0, NCHUNK // U, p6, 0)
            return 0

        lax.cond(jnp.all(nact <= NACT_NAN_MAX), nan_row, full_row, 0)
        pltpu.sync_copy(zbuf, out_hbm.at[row])
        return carry0

    lax.fori_loop(0, ROWS_PER, row_body, 0)


def kernel(logits, mask):
    maskf = mask.astype(jnp.float32)
    return _sc_portfolio(logits, maskf)


# speedup vs baseline: 1.0014x; 1.0014x over previous
import functools

import jax
import jax.numpy as jnp
from jax import lax
from jax.experimental import pallas as pl
from jax.experimental.pallas import tpu as pltpu
from jax.experimental.pallas import tpu_sc as plsc

B = 128
N = 32768
L = 16
NCHUNK = N // L
NBINS = 512
NWORKERS = 32
ROWS_PER = B // NWORKERS
U = 8

F32 = jnp.float32
VN = float(jnp.finfo(jnp.float32).min) / 4.0
BIG = float(jnp.finfo(jnp.float32).max)
NEG_INF = float("-inf")
NAN = float("nan")
NACT_NAN_MAX = float(N // 2 - 5)


def _splat(x):
    return jnp.broadcast_to(x, (L,))


def _vsum(v):
    return _splat(jnp.sum(v))


def _mesh():
    return plsc.VectorSubcoreMesh(core_axis_name="c", subcore_axis_name="s")


@functools.partial(
    pl.kernel,
    out_type=jax.ShapeDtypeStruct((B, N), jnp.float32),
    mesh=_mesh(),
    compiler_params=pltpu.CompilerParams(needs_layout_passes=False),
    scratch_types=[
        pltpu.VMEM((N,), jnp.float32),
        pltpu.VMEM((N,), jnp.float32),
        pltpu.VMEM((NBINS * L,), jnp.float32),
        pltpu.VMEM((NBINS * L,), jnp.float32),
        pltpu.VMEM((NBINS,), jnp.float32),
        pltpu.VMEM((NBINS,), jnp.float32),
    ],
)
def _sc_portfolio(logits_hbm, maskf_hbm, out_hbm, zbuf, mcand, hcnt, hsum,
                  tcnt, tsum):
    wid = lax.axis_index("s") * 2 + lax.axis_index("c")
    lane = lax.iota(jnp.int32, L)
    lanef = lane.astype(F32)
    zeros = jnp.full((L,), 0.0, F32)
    ones = jnp.full((L,), 1.0, F32)

    def row_body(r, carry0):
        row = wid * ROWS_PER + r
        pltpu.sync_copy(logits_hbm.at[row], zbuf)
        pltpu.sync_copy(maskf_hbm.at[row], mcand)

        def p1(jj, carry):
            vmax, vmin, vcnt = carry
            for u in range(U):
                sl = pl.ds((jj * U + u) * L, L)
                v = zbuf[sl]
                m = mcand[sl]
                act = m > 0.0
                z = jnp.where(act, v, jnp.full((L,), VN, F32))
                zbuf[sl] = z
                vmax = jnp.maximum(vmax, z)
                vmin = jnp.minimum(vmin,
                                   jnp.where(act, v, jnp.full((L,), BIG, F32)))
                vcnt = vcnt + jnp.where(act, ones, zeros)
            return vmax, vmin, vcnt

        vmax, vmin, vcnt = lax.fori_loop(
            0, NCHUNK // U, p1,
            (jnp.full((L,), -BIG, F32), jnp.full((L,), BIG, F32), zeros))
        mx = _splat(jnp.max(vmax))
        mn = _splat(jnp.min(vmin))
        nact = _vsum(vcnt)

        def nan_row(_):
            nanv = jnp.full((L,), NAN, F32)

            def pn(jj, _c):
                for u in range(U):
                    zbuf[pl.ds((jj * U + u) * L, L)] = nanv
                return 0
            lax.fori_loop(0, NCHUNK // U, pn, 0)
            return 0

        def full_row(_):
            thr_c = mx - 1.0

            def p2(jj, off):
                for u in range(U):
                    sl = pl.ds((jj * U + u) * L, L)
                    v = zbuf[sl]
                    c = v > thr_c
                    ci = jnp.where(c, jnp.full((L,), 1, jnp.int32),
                                   jnp.full((L,), 0, jnp.int32))
                    pos = plsc.cumsum(ci)
                    idx = pos + (off - 1)
                    plsc.store_scatter(mcand, [idx], v, mask=c)
                    off = off + jnp.sum(ci)
                return off

            nc = lax.fori_loop(0, NCHUNK // U, p2, jnp.int32(0))
            nch = (nc + (L - 1)) >> 4

            def cand_stats(tau):
                def cs(ch, carry):
                    csum, ccnt = carry
                    v = mcand[pl.ds(ch * L, L)]
                    valid = (lane + ch * L) < nc
                    a = jnp.logical_and(valid, v > tau)
                    csum = csum + jnp.where(a, v, zeros)
                    ccnt = ccnt + jnp.where(a, ones, zeros)
                    return csum, ccnt
                s, c = lax.fori_loop(0, nch, cs, (zeros, zeros))
                return _vsum(s), _vsum(c)

            def newton_cond(st):
                _tau, done, it = st
                return jnp.logical_and(jnp.logical_not(done), it < 64)

            def newton_body(st):
                tau, done, it = st
                s, c = cand_stats(tau)
                t2 = (s - 1.0) / jnp.maximum(c, 1.0)
                return t2, jnp.all(t2 == tau), it + 1

            tau_star, _d, _i = lax.while_loop(
                newton_cond, newton_body,
                (mx - 1.0, jnp.bool_(False), jnp.int32(0)))
            _s_unused, kstd = cand_stats(tau_star)

            k = kstd + (float(N - 4) - nact)
            ireg = k - nact

            def finite_tau(_a):
                def hz(jj, _c):
                    for u in range(U):
                        sl = pl.ds((jj * U + u) * L, L)
                        hcnt[sl] = zeros
                        hsum[sl] = zeros
                    return 0
                lax.fori_loop(0, (NBINS * L) // (L * U), hz, 0)

                w = jnp.where(mx > mn, (mx - mn) * (1.0 / float(NBINS)), ones)
                inv_w = 1.0 / w

                def hb(jj, _c):
                    for u in range(U):
                        v = zbuf[pl.ds((jj * U + u) * L, L)]
                        bf = jnp.clip((v - mn) * inv_w, 0.0, float(NBINS - 1))
                        bi = bf.astype(jnp.int32)
                        idx = bi + lane * NBINS
                        plsc.addupdate_scatter(hcnt, [idx], ones)
                        plsc.addupdate_scatter(hsum, [idx], v)
                    return 0
                lax.fori_loop(0, NCHUNK // U, hb, 0)

                def fold(c, _c):
                    sl = pl.ds(c * L, L)
                    acc_c = zeros
                    acc_s = zeros
                    for l in range(L):
                        acc_c = acc_c + hcnt[pl.ds(l * NBINS + c * L, L)]
                        acc_s = acc_s + hsum[pl.ds(l * NBINS + c * L, L)]
                    tcnt[sl] = acc_c
                    tsum[sl] = acc_s
                    return 0
                lax.fori_loop(0, NBINS // L, fold, 0)

                def sw1(tt, carry):
                    base, jbc = carry
                    c = (NBINS // L - 1) - tt
                    t = tcnt[pl.ds(c * L, L)]
                    tot = _vsum(t)
                    suf = base + tot - plsc.cumsum(t) + t
                    jbc = jbc + jnp.where(suf > k, ones, zeros)
                    return base + tot, jbc
                _b, jbc = lax.fori_loop(0, NBINS // L, sw1, (zeros, zeros))
                jbf = _vsum(jbc)

                def sw2(c, carry):
                    cab, sab = carry
                    binf = lanef + _splat(c * L).astype(F32)
                    above = binf > jbf
                    cab = cab + jnp.where(above, tcnt[pl.ds(c * L, L)], zeros)
                    sab = sab + jnp.where(above, tsum[pl.ds(c * L, L)], zeros)
                    return cab, sab
                cab_v, sab_v = lax.fori_loop(0, NBINS // L, sw2,
                                             (zeros, zeros))
                cab = _vsum(cab_v)
                sab = _vsum(sab_v)

                t_edge = mn + jbf * w
                m_rem = k - cab
                sk = sab + m_rem * t_edge
                return (sk - 1.0) / k

            tau_fin = lax.cond(jnp.all(ireg < 0.5), finite_tau,
                               lambda _a: zeros, 0)
            tau_unif = (ireg * F32(VN) - 1.0) / k
            tau = jnp.where(ireg >= 4.5, jnp.full((L,), NEG_INF, F32),
                            jnp.where(ireg >= 0.5, tau_unif, tau_fin))

            def p4(jj, acc):
                for u in range(U):
                    v = zbuf[pl.ds((jj * U + u) * L, L)]
                    acc = acc + jnp.maximum(v - tau, 0.0)
                return acc
            s1 = _vsum(lax.fori_loop(0, NCHUNK // U, p4, zeros))
            r1 = 1.0 / jnp.maximum(s1, 1e-12)

            def p5(jj, acc):
                for u in range(U):
                    v = zbuf[pl.ds((jj * U + u) * L, L)]
                    wv = jnp.maximum(v - tau, 0.0) * r1
                    acc = acc + jnp.where(wv < 1e-6, zeros, wv)
                return acc
            ws = _vsum(lax.fori_loop(0, NCHUNK // U, p5, zeros))
            r2 = 1.0 / jnp.maximum(ws, 1e-12)

            def p6(jj, _c):
                for u in range(U):
                    sl = pl.ds((jj * U + u) * L, L)
                    v = zbuf[sl]
                    wv = jnp.maximum(v - tau, 0.0) * r1
                    wv = jnp.where(wv < 1e-6, zeros, wv)
                    zbuf[sl] = wv * r2
                return 0
            lax.fori_loop(0, NCHUNK // U, p6, 0)
            return 0

        lax.cond(jnp.all(nact <= NACT_NAN_MAX), nan_row, full_row, 0)
        pltpu.sync_copy(zbuf, out_hbm.at[row])
        return carry0

    lax.fori_loop(0, ROWS_PER, row_body, 0)


def kernel(logits, mask):
    maskf = mask.astype(jnp.float32)
    return _sc_portfolio(logits, maskf)


# speedup vs baseline: 1.8062x; 1.8037x over previous
import functools

import jax
import jax.numpy as jnp
from jax import lax
from jax.experimental import pallas as pl
from jax.experimental.pallas import tpu as pltpu
from jax.experimental.pallas import tpu_sc as plsc

B = 128
N = 32768
L = 16
NCHUNK = N // L
NBINS = 512
NCORES = 2
ROWS_PER_CORE = B // NCORES
U = 8

F32 = jnp.float32
VN = float(jnp.finfo(jnp.float32).min) / 4.0
BIG = float(jnp.finfo(jnp.float32).max)
NEG_INF = float("-inf")
NAN = float("nan")
NACT_NAN_MAX = float(N // 2 - 5)


def _splat(x):
    return jnp.broadcast_to(x, (L,))


def _vsum(v):
    return _splat(jnp.sum(v))


def _mesh():
    return plsc.VectorSubcoreMesh(core_axis_name="c", subcore_axis_name="s")


@functools.partial(
    pl.kernel,
    out_type=jax.ShapeDtypeStruct((B, N), jnp.float32),
    mesh=_mesh(),
    compiler_params=pltpu.CompilerParams(needs_layout_passes=False),
    scratch_types=[
        pltpu.VMEM((N,), jnp.float32),
        pltpu.VMEM((N,), jnp.float32),
        pltpu.VMEM((NBINS * L,), jnp.float32),
        pltpu.VMEM((NBINS * L,), jnp.float32),
        pltpu.SMEM((1,), jnp.int32),
    ],
)
def _sc_portfolio(logits_hbm, maskf_hbm, out_hbm, zbuf, mcand, hcnt, hsum,
                  rowcnt):
    sid = lax.axis_index("s")
    cid = lax.axis_index("c")
    lane = lax.iota(jnp.int32, L)
    zeros = jnp.full((L,), 0.0, F32)
    ones = jnp.full((L,), 1.0, F32)

    @pl.when(sid == 0)
    def _init():
        rowcnt[0] = jnp.int32(0)

    plsc.subcore_barrier()

    def process_row(row):
        pltpu.sync_copy(logits_hbm.at[row], zbuf)
        pltpu.sync_copy(maskf_hbm.at[row], mcand)

        def p1(jj, carry):
            vmax, vmin, vcnt = carry
            for u in range(U):
                sl = pl.ds((jj * U + u) * L, L)
                v = zbuf[sl]
                m = mcand[sl]
                act = m > 0.0
                z = jnp.where(act, v, jnp.full((L,), VN, F32))
                zbuf[sl] = z
                vmax = jnp.maximum(vmax, z)
                vmin = jnp.minimum(vmin,
                                   jnp.where(act, v, jnp.full((L,), BIG, F32)))
                vcnt = vcnt + jnp.where(act, ones, zeros)
            return vmax, vmin, vcnt

        vmax, vmin, vcnt = lax.fori_loop(
            0, NCHUNK // U, p1,
            (jnp.full((L,), -BIG, F32), jnp.full((L,), BIG, F32), zeros))
        mx = _splat(jnp.max(vmax))
        mn = _splat(jnp.min(vmin))
        nact = _vsum(vcnt)

        def nan_row(_):
            nanv = jnp.full((L,), NAN, F32)

            def pn(jj, _c):
                for u in range(U):
                    zbuf[pl.ds((jj * U + u) * L, L)] = nanv
                return 0
            lax.fori_loop(0, NCHUNK // U, pn, 0)
            return 0

        def full_row(_):
            thr_c = mx - 1.0

            def p2(jj, off):
                for u in range(U):
                    sl = pl.ds((jj * U + u) * L, L)
                    v = zbuf[sl]
                    c = v > thr_c
                    ci = jnp.where(c, jnp.full((L,), 1, jnp.int32),
                                   jnp.full((L,), 0, jnp.int32))
                    pos = plsc.cumsum(ci)
                    idx = pos + (off - 1)
                    plsc.store_scatter(mcand, [idx], v, mask=c)
                    off = off + jnp.sum(ci)
                return off

            nc = lax.fori_loop(0, NCHUNK // U, p2, jnp.int32(0))
            nch = (nc + (L - 1)) >> 4

            def cand_stats(tau):
                def cs(ch, carry):
                    csum, ccnt = carry
                    v = mcand[pl.ds(ch * L, L)]
                    valid = (lane + ch * L) < nc
                    a = jnp.logical_and(valid, v > tau)
                    csum = csum + jnp.where(a, v, zeros)
                    ccnt = ccnt + jnp.where(a, ones, zeros)
                    return csum, ccnt
                s, c = lax.fori_loop(0, nch, cs, (zeros, zeros))
                return _vsum(s), _vsum(c)

            def newton_cond(st):
                _tau, done, it = st
                return jnp.logical_and(jnp.logical_not(done), it < 64)

            def newton_body(st):
                tau, done, it = st
                s, c = cand_stats(tau)
                t2 = (s - 1.0) / jnp.maximum(c, 1.0)
                return t2, jnp.all(t2 == tau), it + 1

            tau_star, _d, _i = lax.while_loop(
                newton_cond, newton_body,
                (mx - 1.0, jnp.bool_(False), jnp.int32(0)))
            _s_unused, kstd = cand_stats(tau_star)

            k = kstd + (float(N - 4) - nact)
            ireg = k - nact

            def finite_tau(_a):
                def hz(jj, _c):
                    for u in range(U):
                        sl = pl.ds((jj * U + u) * L, L)
                        hcnt[sl] = zeros
                        hsum[sl] = zeros
                    return 0
                lax.fori_loop(0, (NBINS * L) // (L * U), hz, 0)

                w = jnp.where(mx > mn, (mx - mn) * (1.0 / float(NBINS)), ones)
                inv_w = 1.0 / w

                def hb(jj, _c):
                    for u in range(U):
                        v = zbuf[pl.ds((jj * U + u) * L, L)]
                        bf = jnp.clip((v - mn) * inv_w, 0.0, float(NBINS - 1))
                        bi = bf.astype(jnp.int32)
                        idx = bi * L + lane
                        plsc.addupdate_scatter(hcnt, [idx], ones)
                        plsc.addupdate_scatter(hsum, [idx], v)
                    return 0
                lax.fori_loop(0, NCHUNK // U, hb, 0)

                def sw1(tt, carry):
                    run, jbc = carry
                    for u in range(U):
                        bb = (NBINS - 1) - (tt * U + u)
                        tb = _vsum(hcnt[pl.ds(bb * L, L)])
                        run = run + tb
                        jbc = jbc + jnp.where(run > k, ones, zeros)
                    return run, jbc
                _run, jbf = lax.fori_loop(0, NBINS // U, sw1, (zeros, zeros))

                def sw2(tt, carry):
                    cab, sab = carry
                    for u in range(U):
                        bb = (NBINS - 1) - (tt * U + u)
                        above = _splat(jnp.int32(bb)).astype(F32) > jbf
                        tb = _vsum(hcnt[pl.ds(bb * L, L)])
                        ts = _vsum(hsum[pl.ds(bb * L, L)])
                        cab = cab + jnp.where(above, tb, zeros)
                        sab = sab + jnp.where(above, ts, zeros)
                    return cab, sab
                cab, sab = lax.fori_loop(0, NBINS // U, sw2, (zeros, zeros))

                t_edge = mn + jbf * w
                m_rem = k - cab
                sk = sab + m_rem * t_edge
                return (sk - 1.0) / k

            tau_fin = lax.cond(jnp.all(ireg < 0.5), finite_tau,
                               lambda _a: zeros, 0)
            tau_unif = (ireg * F32(VN) - 1.0) / k
            tau = jnp.where(ireg >= 4.5, jnp.full((L,), NEG_INF, F32),
                            jnp.where(ireg >= 0.5, tau_unif, tau_fin))

            def p4(jj, acc):
                for u in range(U):
                    v = zbuf[pl.ds((jj * U + u) * L, L)]
                    acc = acc + jnp.maximum(v - tau, 0.0)
                return acc
            s1 = _vsum(lax.fori_loop(0, NCHUNK // U, p4, zeros))
            r1 = 1.0 / jnp.maximum(s1, 1e-12)

            def p5(jj, acc):
                for u in range(U):
                    sl = pl.ds((jj * U + u) * L, L)
                    v = zbuf[sl]
                    p = jnp.maximum(v - tau, 0.0)
                    wv = p * r1
                    wv = jnp.where(wv < 1e-6, zeros, wv)
                    zbuf[sl] = wv
                    acc = acc + wv
                return acc
            ws = _vsum(lax.fori_loop(0, NCHUNK // U, p5, zeros))
            r2 = 1.0 / jnp.maximum(ws, 1e-12)

            def p6(jj, _c):
                for u in range(U):
                    sl = pl.ds((jj * U + u) * L, L)
                    zbuf[sl] = zbuf[sl] * r2
                return 0
            lax.fori_loop(0, NCHUNK // U, p6, 0)
            return 0

        lax.cond(jnp.all(nact <= NACT_NAN_MAX), nan_row, full_row, 0)
        pltpu.sync_copy(zbuf, out_hbm.at[row])

    def work_cond(idx):
        return idx < ROWS_PER_CORE

    def work_body(idx):
        process_row(cid * ROWS_PER_CORE + idx)
        return plsc.fetch_and_add(rowcnt.at[0], jnp.int32(1), subcore_id=0)

    idx0 = plsc.fetch_and_add(rowcnt.at[0], jnp.int32(1), subcore_id=0)
    lax.while_loop(work_cond, work_body, idx0)


def kernel(logits, mask):
    maskf = mask.astype(jnp.float32)
    return _sc_portfolio(logits, maskf)


# speedup vs baseline: 1.8567x; 1.0280x over previous
import functools

import jax
import jax.numpy as jnp
from jax import lax
from jax.experimental import pallas as pl
from jax.experimental.pallas import tpu as pltpu
from jax.experimental.pallas import tpu_sc as plsc

B = 128
N = 32768
L = 16
NCHUNK = N // L
NBINS = 256
NCORES = 2
ROWS_PER_CORE = B // NCORES
U = 16

F32 = jnp.float32
VN = float(jnp.finfo(jnp.float32).min) / 4.0
BIG = float(jnp.finfo(jnp.float32).max)
NEG_INF = float("-inf")
NAN = float("nan")
NACT_NAN_MAX = float(N // 2 - 5)


def _splat(x):
    return jnp.broadcast_to(x, (L,))


def _vsum(v):
    return _splat(jnp.sum(v))


def _mesh():
    return plsc.VectorSubcoreMesh(core_axis_name="c", subcore_axis_name="s")


@functools.partial(
    pl.kernel,
    out_type=jax.ShapeDtypeStruct((B, N), jnp.float32),
    mesh=_mesh(),
    compiler_params=pltpu.CompilerParams(needs_layout_passes=False),
    scratch_types=[
        pltpu.VMEM((N,), jnp.float32),
        pltpu.VMEM((N,), jnp.float32),
        pltpu.VMEM((NBINS * L,), jnp.float32),
        pltpu.VMEM((NBINS * L,), jnp.float32),
        pltpu.SMEM((1,), jnp.int32),
    ],
)
def _sc_portfolio(logits_hbm, maskf_hbm, out_hbm, zbuf, mcand, hcnt, hsum,
                  rowcnt):
    sid = lax.axis_index("s")
    cid = lax.axis_index("c")
    lane = lax.iota(jnp.int32, L)
    zeros = jnp.full((L,), 0.0, F32)
    ones = jnp.full((L,), 1.0, F32)

    @pl.when(sid == 0)
    def _init():
        rowcnt[0] = jnp.int32(0)

    plsc.subcore_barrier()

    def process_row(row):
        pltpu.sync_copy(logits_hbm.at[row], zbuf)
        pltpu.sync_copy(maskf_hbm.at[row], mcand)

        def p1(jj, carry):
            vmax, vmin, vcnt = carry
            for u in range(U):
                sl = pl.ds((jj * U + u) * L, L)
                v = zbuf[sl]
                m = mcand[sl]
                act = m > 0.0
                z = jnp.where(act, v, jnp.full((L,), VN, F32))
                zbuf[sl] = z
                vmax = jnp.maximum(vmax, z)
                vmin = jnp.minimum(vmin,
                                   jnp.where(act, v, jnp.full((L,), BIG, F32)))
                vcnt = vcnt + jnp.where(act, ones, zeros)
            return vmax, vmin, vcnt

        vmax, vmin, vcnt = lax.fori_loop(
            0, NCHUNK // U, p1,
            (jnp.full((L,), -BIG, F32), jnp.full((L,), BIG, F32), zeros))
        mx = _splat(jnp.max(vmax))
        mn = _splat(jnp.min(vmin))
        nact = _vsum(vcnt)

        def nan_row(_):
            nanv = jnp.full((L,), NAN, F32)

            def pn(jj, _c):
                for u in range(U):
                    zbuf[pl.ds((jj * U + u) * L, L)] = nanv
                return 0
            lax.fori_loop(0, NCHUNK // U, pn, 0)
            return 0

        def full_row(_):
            thr_c = mx - 1.0

            def p2(jj, off):
                for u in range(U):
                    sl = pl.ds((jj * U + u) * L, L)
                    v = zbuf[sl]
                    c = v > thr_c
                    ci = jnp.where(c, jnp.full((L,), 1, jnp.int32),
                                   jnp.full((L,), 0, jnp.int32))
                    pos = plsc.cumsum(ci)
                    idx = pos + (off - 1)
                    plsc.store_scatter(mcand, [idx], v, mask=c)
                    off = off + jnp.sum(ci)
                return off

            nc = lax.fori_loop(0, NCHUNK // U, p2, jnp.int32(0))
            nch = (nc + (L - 1)) >> 4

            def cand_stats(tau):
                def cs(ch, carry):
                    csum, ccnt = carry
                    v = mcand[pl.ds(ch * L, L)]
                    valid = (lane + ch * L) < nc
                    a = jnp.logical_and(valid, v > tau)
                    csum = csum + jnp.where(a, v, zeros)
                    ccnt = ccnt + jnp.where(a, ones, zeros)
                    return csum, ccnt
                s, c = lax.fori_loop(0, nch, cs, (zeros, zeros))
                return _vsum(s), _vsum(c)

            def newton_cond(st):
                _tau, done, it = st
                return jnp.logical_and(jnp.logical_not(done), it < 64)

            def newton_body(st):
                tau, done, it = st
                s, c = cand_stats(tau)
                t2 = (s - 1.0) / jnp.maximum(c, 1.0)
                return t2, jnp.all(t2 == tau), it + 1

            tau_star, _d, _i = lax.while_loop(
                newton_cond, newton_body,
                (mx - 1.0, jnp.bool_(False), jnp.int32(0)))
            _s_unused, kstd = cand_stats(tau_star)

            k = kstd + (float(N - 4) - nact)
            ireg = k - nact

            def finite_tau(_a):
                def hz(jj, _c):
                    for u in range(U):
                        sl = pl.ds((jj * U + u) * L, L)
                        hcnt[sl] = zeros
                        hsum[sl] = zeros
                    return 0
                lax.fori_loop(0, (NBINS * L) // (L * U), hz, 0)

                w = jnp.where(mx > mn, (mx - mn) * (1.0 / float(NBINS)), ones)
                inv_w = 1.0 / w

                def hb(jj, _c):
                    for u in range(U):
                        v = zbuf[pl.ds((jj * U + u) * L, L)]
                        bf = jnp.clip((v - mn) * inv_w, 0.0, float(NBINS - 1))
                        bi = bf.astype(jnp.int32)
                        idx = bi * L + lane
                        plsc.addupdate_scatter(hcnt, [idx], ones)
                        plsc.addupdate_scatter(hsum, [idx], v)
                    return 0
                lax.fori_loop(0, NCHUNK // U, hb, 0)

                def sw1(tt, carry):
                    run, jbc = carry
                    for u in range(U):
                        bb = (NBINS - 1) - (tt * U + u)
                        tb = _vsum(hcnt[pl.ds(bb * L, L)])
                        run = run + tb
                        jbc = jbc + jnp.where(run > k, ones, zeros)
                    return run, jbc
                _run, jbf = lax.fori_loop(0, NBINS // U, sw1, (zeros, zeros))

                def sw2(tt, carry):
                    cab, sab = carry
                    for u in range(U):
                        bb = (NBINS - 1) - (tt * U + u)
                        above = _splat(jnp.int32(bb)).astype(F32) > jbf
                        tb = _vsum(hcnt[pl.ds(bb * L, L)])
                        ts = _vsum(hsum[pl.ds(bb * L, L)])
                        cab = cab + jnp.where(above, tb, zeros)
                        sab = sab + jnp.where(above, ts, zeros)
                    return cab, sab
                cab, sab = lax.fori_loop(0, NBINS // U, sw2, (zeros, zeros))

                t_edge = mn + jbf * w
                m_rem = k - cab
                sk = sab + m_rem * t_edge
                return (sk - 1.0) / k

            tau_fin = lax.cond(jnp.all(ireg < 0.5), finite_tau,
                               lambda _a: zeros, 0)
            tau_unif = (ireg * F32(VN) - 1.0) / k
            tau = jnp.where(ireg >= 4.5, jnp.full((L,), NEG_INF, F32),
                            jnp.where(ireg >= 0.5, tau_unif, tau_fin))

            def p4(jj, acc):
                for u in range(U):
                    v = zbuf[pl.ds((jj * U + u) * L, L)]
                    acc = acc + jnp.maximum(v - tau, 0.0)
                return acc
            s1 = _vsum(lax.fori_loop(0, NCHUNK // U, p4, zeros))
            r1 = 1.0 / jnp.maximum(s1, 1e-12)

            def p5(jj, acc):
                for u in range(U):
                    sl = pl.ds((jj * U + u) * L, L)
                    v = zbuf[sl]
                    p = jnp.maximum(v - tau, 0.0)
                    wv = p * r1
                    wv = jnp.where(wv < 1e-6, zeros, wv)
                    zbuf[sl] = wv
                    acc = acc + wv
                return acc
            ws = _vsum(lax.fori_loop(0, NCHUNK // U, p5, zeros))
            r2 = 1.0 / jnp.maximum(ws, 1e-12)

            def p6(jj, _c):
                for u in range(U):
                    sl = pl.ds((jj * U + u) * L, L)
                    zbuf[sl] = zbuf[sl] * r2
                return 0
            lax.fori_loop(0, NCHUNK // U, p6, 0)
            return 0

        lax.cond(jnp.all(nact <= NACT_NAN_MAX), nan_row, full_row, 0)
        pltpu.sync_copy(zbuf, out_hbm.at[row])

    def work_cond(idx):
        return idx < ROWS_PER_CORE

    def work_body(idx):
        process_row(cid * ROWS_PER_CORE + idx)
        return plsc.fetch_and_add(rowcnt.at[0], jnp.int32(1), subcore_id=0)

    idx0 = plsc.fetch_and_add(rowcnt.at[0], jnp.int32(1), subcore_id=0)
    lax.while_loop(work_cond, work_body, idx0)


def kernel(logits, mask):
    maskf = mask.astype(jnp.float32)
    return _sc_portfolio(logits, maskf)


# speedup vs baseline: 2.1083x; 1.1355x over previous
import functools

import jax
import jax.numpy as jnp
from jax import lax
from jax.experimental import pallas as pl
from jax.experimental.pallas import tpu as pltpu
from jax.experimental.pallas import tpu_sc as plsc

B = 128
N = 32768
L = 16
NCHUNK = N // L
NBINS = 256
NCORES = 2
ROWS_PER_CORE = B // NCORES
U = 16

F32 = jnp.float32
VN = float(jnp.finfo(jnp.float32).min) / 4.0
BIG = float(jnp.finfo(jnp.float32).max)
NEG_INF = float("-inf")
NAN = float("nan")
NACT_NAN_MAX = float(N // 2 - 5)


def _splat(x):
    return jnp.broadcast_to(x, (L,))


def _vsum(v):
    return _splat(jnp.sum(v))


def _mesh():
    return plsc.VectorSubcoreMesh(core_axis_name="c", subcore_axis_name="s")


@functools.partial(
    pl.kernel,
    out_type=jax.ShapeDtypeStruct((B, N), jnp.float32),
    mesh=_mesh(),
    compiler_params=pltpu.CompilerParams(needs_layout_passes=False),
    scratch_types=[
        pltpu.VMEM((N,), jnp.float32),
        pltpu.VMEM((N,), jnp.float32),
        pltpu.VMEM((NBINS * L,), jnp.float32),
        pltpu.VMEM((NBINS * L,), jnp.float32),
        pltpu.VMEM(((N // (L * U)) * L,), jnp.float32),
        pltpu.SMEM((1,), jnp.int32),
    ],
)
def _sc_portfolio(logits_hbm, maskf_hbm, out_hbm, zbuf, mcand, hcnt, hsum,
                  gmax, rowcnt):
    sid = lax.axis_index("s")
    cid = lax.axis_index("c")
    lane = lax.iota(jnp.int32, L)
    zeros = jnp.full((L,), 0.0, F32)
    ones = jnp.full((L,), 1.0, F32)

    @pl.when(sid == 0)
    def _init():
        rowcnt[0] = jnp.int32(0)

    plsc.subcore_barrier()

    def process_row(row):
        pltpu.sync_copy(logits_hbm.at[row], zbuf)
        pltpu.sync_copy(maskf_hbm.at[row], mcand)

        def p1(jj, carry):
            vmax, vmin, vcnt = carry
            bmax = jnp.full((L,), -BIG, F32)
            for u in range(U):
                sl = pl.ds((jj * U + u) * L, L)
                v = zbuf[sl]
                m = mcand[sl]
                act = m > 0.0
                z = jnp.where(act, v, jnp.full((L,), VN, F32))
                zbuf[sl] = z
                bmax = jnp.maximum(bmax, z)
                vmin = jnp.minimum(vmin,
                                   jnp.where(act, v, jnp.full((L,), BIG, F32)))
                vcnt = vcnt + jnp.where(act, ones, zeros)
            gmax[pl.ds(jj * L, L)] = bmax
            vmax = jnp.maximum(vmax, bmax)
            return vmax, vmin, vcnt

        vmax, vmin, vcnt = lax.fori_loop(
            0, NCHUNK // U, p1,
            (jnp.full((L,), -BIG, F32), jnp.full((L,), BIG, F32), zeros))
        mx = _splat(jnp.max(vmax))
        mn = _splat(jnp.min(vmin))
        nact = _vsum(vcnt)

        def nan_row(_):
            nanv = jnp.full((L,), NAN, F32)

            def pn(jj, _c):
                for u in range(U):
                    zbuf[pl.ds((jj * U + u) * L, L)] = nanv
                return 0
            lax.fori_loop(0, NCHUNK // U, pn, 0)
            return 0

        def full_row(_):
            thr_c = mx - 1.0

            def p2(jj, off):
                gv = gmax[pl.ds(jj * L, L)]

                def scan_body(off_in):
                    for u in range(U):
                        sl = pl.ds((jj * U + u) * L, L)
                        v = zbuf[sl]
                        c = v > thr_c
                        ci = jnp.where(c, jnp.full((L,), 1, jnp.int32),
                                       jnp.full((L,), 0, jnp.int32))
                        pos = plsc.cumsum(ci)
                        idx = pos + (off_in - 1)
                        plsc.store_scatter(mcand, [idx], v, mask=c)
                        off_in = off_in + jnp.sum(ci)
                    return off_in

                return lax.cond(jnp.any(gv > thr_c), scan_body,
                                lambda o: o, off)

            nc = lax.fori_loop(0, NCHUNK // U, p2, jnp.int32(0))
            nch = (nc + (L - 1)) >> 4

            def cand_stats(tau):
                def cs(ch, carry):
                    csum, ccnt = carry
                    v = mcand[pl.ds(ch * L, L)]
                    valid = (lane + ch * L) < nc
                    a = jnp.logical_and(valid, v > tau)
                    csum = csum + jnp.where(a, v, zeros)
                    ccnt = ccnt + jnp.where(a, ones, zeros)
                    return csum, ccnt
                s, c = lax.fori_loop(0, nch, cs, (zeros, zeros))
                return _vsum(s), _vsum(c)

            def newton_cond(st):
                _tau, done, it = st
                return jnp.logical_and(jnp.logical_not(done), it < 64)

            def newton_body(st):
                tau, done, it = st
                s, c = cand_stats(tau)
                t2 = (s - 1.0) / jnp.maximum(c, 1.0)
                return t2, jnp.all(t2 == tau), it + 1

            tau_star, _d, _i = lax.while_loop(
                newton_cond, newton_body,
                (mx - 1.0, jnp.bool_(False), jnp.int32(0)))
            _s_unused, kstd = cand_stats(tau_star)

            k = kstd + (float(N - 4) - nact)
            ireg = k - nact

            def finite_tau(_a):
                def hz(jj, _c):
                    for u in range(U):
                        sl = pl.ds((jj * U + u) * L, L)
                        hcnt[sl] = zeros
                        hsum[sl] = zeros
                    return 0
                lax.fori_loop(0, (NBINS * L) // (L * U), hz, 0)

                w = jnp.where(mx > mn, (mx - mn) * (1.0 / float(NBINS)), ones)
                inv_w = 1.0 / w

                def hb(jj, _c):
                    for u in range(U):
                        v = zbuf[pl.ds((jj * U + u) * L, L)]
                        bf = jnp.clip((v - mn) * inv_w, 0.0, float(NBINS - 1))
                        bi = bf.astype(jnp.int32)
                        idx = bi * L + lane
                        plsc.addupdate_scatter(hcnt, [idx], ones)
                        plsc.addupdate_scatter(hsum, [idx], v)
                    return 0
                lax.fori_loop(0, NCHUNK // U, hb, 0)

                def sw1(tt, carry):
                    run, jbc = carry
                    for u in range(U):
                        bb = (NBINS - 1) - (tt * U + u)
                        tb = _vsum(hcnt[pl.ds(bb * L, L)])
                        run = run + tb
                        jbc = jbc + jnp.where(run > k, ones, zeros)
                    return run, jbc
                _run, jbf = lax.fori_loop(0, NBINS // U, sw1, (zeros, zeros))

                def sw2(tt, carry):
                    cab, sab = carry
                    for u in range(U):
                        bb = (NBINS - 1) - (tt * U + u)
                        above = _splat(jnp.int32(bb)).astype(F32) > jbf
                        tb = _vsum(hcnt[pl.ds(bb * L, L)])
                        ts = _vsum(hsum[pl.ds(bb * L, L)])
                        cab = cab + jnp.where(above, tb, zeros)
                        sab = sab + jnp.where(above, ts, zeros)
                    return cab, sab
                cab, sab = lax.fori_loop(0, NBINS // U, sw2, (zeros, zeros))

                t_edge = mn + jbf * w
                m_rem = k - cab
                sk = sab + m_rem * t_edge
                return (sk - 1.0) / k

            tau_fin = lax.cond(jnp.all(ireg < 0.5), finite_tau,
                               lambda _a: zeros, 0)
            tau_unif = (ireg * F32(VN) - 1.0) / k
            tau = jnp.where(ireg >= 4.5, jnp.full((L,), NEG_INF, F32),
                            jnp.where(ireg >= 0.5, tau_unif, tau_fin))

            def p4(jj, acc):
                for u in range(U):
                    v = zbuf[pl.ds((jj * U + u) * L, L)]
                    acc = acc + jnp.maximum(v - tau, 0.0)
                return acc
            s1 = _vsum(lax.fori_loop(0, NCHUNK // U, p4, zeros))
            r1 = 1.0 / jnp.maximum(s1, 1e-12)

            def p5(jj, acc):
                for u in range(U):
                    sl = pl.ds((jj * U + u) * L, L)
                    v = zbuf[sl]
                    p = jnp.maximum(v - tau, 0.0)
                    wv = p * r1
                    wv = jnp.where(wv < 1e-6, zeros, wv)
                    zbuf[sl] = wv
                    acc = acc + wv
                return acc
            ws = _vsum(lax.fori_loop(0, NCHUNK // U, p5, zeros))
            r2 = 1.0 / jnp.maximum(ws, 1e-12)

            def p6(jj, _c):
                for u in range(U):
                    sl = pl.ds((jj * U + u) * L, L)
                    zbuf[sl] = zbuf[sl] * r2
                return 0
            lax.fori_loop(0, NCHUNK // U, p6, 0)
            return 0

        lax.cond(jnp.all(nact <= NACT_NAN_MAX), nan_row, full_row, 0)
        pltpu.sync_copy(zbuf, out_hbm.at[row])

    def work_cond(idx):
        return idx < ROWS_PER_CORE

    def work_body(idx):
        process_row(cid * ROWS_PER_CORE + idx)
        return plsc.fetch_and_add(rowcnt.at[0], jnp.int32(1), subcore_id=0)

    idx0 = plsc.fetch_and_add(rowcnt.at[0], jnp.int32(1), subcore_id=0)
    lax.while_loop(work_cond, work_body, idx0)


def kernel(logits, mask):
    maskf = mask.astype(jnp.float32)
    return _sc_portfolio(logits, maskf)
